# Initial kernel scaffold; baseline (speedup 1.0000x reference)
import jax
import jax.numpy as jnp
from jax.experimental import pallas as pl


def kernel(seq, W, bias):
    raise NotImplementedError("write your pallas kernel here")


# speedup vs baseline: 1.0605x; 1.0605x over previous
import functools

import jax
import jax.numpy as jnp
from jax import lax
from jax.experimental import pallas as pl
from jax.experimental.pallas import tpu as pltpu

_B, _N, _E = 4, 8192, 768
_M = 64
_TOP_K = 8
_EPS = 1e-06

_TB = 512


def _router_body(wt_ref, bias_ref, x_ref, logits_ref, idx_ref, w_ref):
    x = x_ref[...]
    wt = wt_ref[...]
    logits = jnp.dot(x, wt, preferred_element_type=jnp.float32)
    logits_ref[...] = logits

    m = jnp.max(logits, axis=-1, keepdims=True)
    ex = jnp.exp(logits - m)
    probs = ex / jnp.sum(ex, axis=-1, keepdims=True)
    adj = probs + bias_ref[...]

    iota = lax.broadcasted_iota(jnp.int32, (_TB, _M), 1)
    work = adj
    idxs, ws = [], []
    for _ in range(_TOP_K):
        mx = jnp.max(work, axis=-1, keepdims=True)
        ik = jnp.min(jnp.where(work == mx, iota, _M), axis=-1, keepdims=True)
        hit = iota == ik
        wk = jnp.sum(jnp.where(hit, probs, 0.0), axis=-1, keepdims=True)
        work = jnp.where(hit, -jnp.inf, work)
        idxs.append(ik)
        ws.append(wk)

    idx = jnp.concatenate(idxs, axis=-1)
    wv = jnp.concatenate(ws, axis=-1)
    wv = wv / (jnp.sum(wv, axis=-1, keepdims=True) + _EPS)
    idx_ref[...] = idx
    w_ref[...] = wv


@jax.jit
def _router(seq2d, wt, bias2d):
    n_tok = seq2d.shape[0]
    grid = (n_tok // _TB,)
    return pl.pallas_call(
        _router_body,
        grid=grid,
        in_specs=[
            pl.BlockSpec((_E, _M), lambda i: (0, 0)),
            pl.BlockSpec((1, _M), lambda i: (0, 0)),
            pl.BlockSpec((_TB, _E), lambda i: (i, 0)),
        ],
        out_specs=[
            pl.BlockSpec((_TB, _M), lambda i: (i, 0)),
            pl.BlockSpec((_TB, _TOP_K), lambda i: (i, 0)),
            pl.BlockSpec((_TB, _TOP_K), lambda i: (i, 0)),
        ],
        out_shape=[
            jax.ShapeDtypeStruct((n_tok, _M), jnp.float32),
            jax.ShapeDtypeStruct((n_tok, _TOP_K), jnp.int32),
            jax.ShapeDtypeStruct((n_tok, _TOP_K), jnp.float32),
        ],
    )(wt, bias2d, seq2d)


def kernel(seq, W, bias):
    b, n, e = seq.shape
    seq2d = seq.reshape(b * n, e)
    wt = W.T
    bias2d = bias.reshape(1, _M)
    logits, idx, wv = _router(seq2d, wt, bias2d)
    return (
        logits.reshape(b, n, _M),
        idx.reshape(b, n, _TOP_K),
        seq,
        wv.reshape(b, n, _TOP_K),
    )


# speedup vs baseline: 1.3275x; 1.2518x over previous
import functools

import jax
import jax.numpy as jnp
from jax import lax
from jax.experimental import pallas as pl
from jax.experimental.pallas import tpu as pltpu

_B, _N, _E = 4, 8192, 768
_M = 64
_TOP_K = 8
_EPS = 1e-06

_TB = 512


def _router_body(wt_ref, bias_ref, x_ref, logits_ref, idx_ref, w_ref, seq_out_ref):
    x = x_ref[...]
    seq_out_ref[...] = x
    wt = wt_ref[...]
    logits = jnp.dot(x, wt, preferred_element_type=jnp.float32)
    logits_ref[...] = logits

    m = jnp.max(logits, axis=-1, keepdims=True)
    ex = jnp.exp(logits - m)
    probs = ex / jnp.sum(ex, axis=-1, keepdims=True)
    adj = probs + bias_ref[...]

    iota = lax.broadcasted_iota(jnp.int32, (_TB, _M), 1)
    work = adj
    idxs, ws = [], []
    for _ in range(_TOP_K):
        mx = jnp.max(work, axis=-1, keepdims=True)
        ik = jnp.min(jnp.where(work == mx, iota, _M), axis=-1, keepdims=True)
        hit = iota == ik
        wk = jnp.sum(jnp.where(hit, probs, 0.0), axis=-1, keepdims=True)
        work = jnp.where(hit, -jnp.inf, work)
        idxs.append(ik)
        ws.append(wk)

    idx = jnp.concatenate(idxs, axis=-1)
    wv = jnp.concatenate(ws, axis=-1)
    wv = wv / (jnp.sum(wv, axis=-1, keepdims=True) + _EPS)
    idx_ref[...] = idx
    w_ref[...] = wv


@jax.jit
def _router(seq2d, wt, bias2d):
    n_tok = seq2d.shape[0]
    grid = (n_tok // _TB,)
    return pl.pallas_call(
        _router_body,
        grid=grid,
        in_specs=[
            pl.BlockSpec((_E, _M), lambda i: (0, 0)),
            pl.BlockSpec((1, _M), lambda i: (0, 0)),
            pl.BlockSpec((_TB, _E), lambda i: (i, 0)),
        ],
        out_specs=[
            pl.BlockSpec((_TB, _M), lambda i: (i, 0)),
            pl.BlockSpec((_TB, _TOP_K), lambda i: (i, 0)),
            pl.BlockSpec((_TB, _TOP_K), lambda i: (i, 0)),
            pl.BlockSpec((_TB, _E), lambda i: (i, 0)),
        ],
        out_shape=[
            jax.ShapeDtypeStruct((n_tok, _M), jnp.float32),
            jax.ShapeDtypeStruct((n_tok, _TOP_K), jnp.int32),
            jax.ShapeDtypeStruct((n_tok, _TOP_K), jnp.float32),
            jax.ShapeDtypeStruct((n_tok, _E), jnp.float32),
        ],
    )(wt, bias2d, seq2d)


def kernel(seq, W, bias):
    b, n, e = seq.shape
    seq2d = seq.reshape(b * n, e)
    wt = W.T
    bias2d = bias.reshape(1, _M)
    logits, idx, wv, seq_out = _router(seq2d, wt, bias2d)
    return (
        logits.reshape(b, n, _M),
        idx.reshape(b, n, _TOP_K),
        seq_out.reshape(b, n, e),
        wv.reshape(b, n, _TOP_K),
    )
